# baseline (speedup 1.0000x reference)
import functools

import jax
import jax.numpy as jnp
from jax import lax
from jax.experimental import pallas as pl
from jax.experimental.pallas import tpu as pltpu
from jax.experimental.pallas import tpu_sc as plsc

VOCAB = 1000
VPAD = 1024
NC, NS, L = 2, 16, 16
NW = NC * NS
N_TOK = 16 * 2048
TPW = N_TOK // NW
C = 16
NCH = TPW // C
CW = C * VOCAB


def _lse_pad_body(table_ref, lse_ref, padded_ref):
    x = table_ref[...]
    m = jnp.max(x, axis=1, keepdims=True)
    s = jnp.sum(jnp.exp(x - m), axis=1, keepdims=True)
    lse_ref[...] = m + jnp.log(s)
    padded_ref[...] = jnp.concatenate(
        [x, jnp.zeros((VOCAB, VPAD - VOCAB), jnp.float32)], axis=1)


def _lse_and_pad(table):
    lse2d, padded = pl.pallas_call(
        _lse_pad_body,
        out_shape=(
            jax.ShapeDtypeStruct((VOCAB, 1), jnp.float32),
            jax.ShapeDtypeStruct((VOCAB, VPAD), jnp.float32),
        ),
    )(table)
    return (jnp.pad(lse2d.reshape(VOCAB), (0, VPAD - VOCAB)),
            padded.reshape(VOCAB, VPAD // 128, 128))


def _sc_body(idx_hbm, tgt_hbm, lse_hbm, table_hbm, out_hbm, loss_hbm,
             idx_v, tgt_v, lse_v, g_a, g_b, p_a, p_b, acc_v,
             sem_ga, sem_gb, sem_sa, sem_sb):
    wid = lax.axis_index("c") * NS + lax.axis_index("s")
    base = wid * TPW
    pltpu.sync_copy(idx_hbm.at[pl.ds(base, TPW)], idx_v)
    pltpu.sync_copy(tgt_hbm.at[pl.ds(base, TPW)], tgt_v)
    pltpu.sync_copy(lse_hbm, lse_v)
    lane = lax.iota(jnp.int32, L)
    tail_mask = lane < (VOCAB % L)

    def chunk_ids(c):
        return idx_v[pl.ds(c * C, C)]

    def gather_start(c, g, sem):
        pltpu.async_copy(table_hbm.at[chunk_ids(c)], g, sem)

    def gather_wait(c, g, sem):
        pltpu.make_async_copy(table_hbm.at[chunk_ids(c)], g, sem).wait()

    def store_start(c, p, sem):
        pltpu.async_copy(p, out_hbm.at[pl.ds((base + c * C) * VOCAB, CW)],
                         sem)

    def store_drain(c, p, sem):
        pltpu.make_async_copy(
            p, out_hbm.at[pl.ds((base + c * C) * VOCAB, CW)], sem).wait()

    def repack(g, p):
        for j in range(C):
            dbase = VOCAB * j
            for k in range(VOCAB // L):
                v = g[j, k // 8, pl.ds((k % 8) * L, L)]
                plsc.store_scatter(p, [lane + (dbase + L * k)], v)
            v = g[j, 7, pl.ds(96, L)]
            plsc.store_scatter(p, [lane + (dbase + VOCAB - (VOCAB % L))], v,
                               mask=tail_mask)

    def loss_chunk(c, g, acc):
        ids = chunk_ids(c)
        tgs = tgt_v[pl.ds(c * C, C)]
        ls = plsc.load_gather(lse_v, [ids])
        tl = plsc.load_gather(g, [lane, tgs >> 7, tgs & 127])
        return acc + (ls - tl)

    gather_start(0, g_a, sem_ga)

    def body(i2, acc):
        ca = 2 * i2
        cb = ca + 1

        gather_start(cb, g_b, sem_gb)
        gather_wait(ca, g_a, sem_ga)
        acc = loss_chunk(ca, g_a, acc)

        @pl.when(i2 > 0)
        def _():
            store_drain(ca - 2, p_a, sem_sa)

        repack(g_a, p_a)
        store_start(ca, p_a, sem_sa)

        @pl.when(ca + 2 < NCH)
        def _():
            gather_start(ca + 2, g_a, sem_ga)

        gather_wait(cb, g_b, sem_gb)
        acc = loss_chunk(cb, g_b, acc)

        @pl.when(i2 > 0)
        def _():
            store_drain(cb - 2, p_b, sem_sb)

        repack(g_b, p_b)
        store_start(cb, p_b, sem_sb)
        return acc

    acc = lax.fori_loop(0, NCH // 2, body, jnp.zeros((L,), jnp.float32))
    store_drain(NCH - 2, p_a, sem_sa)
    store_drain(NCH - 1, p_b, sem_sb)
    acc_v[pl.ds(0, L)] = acc
    zeros = jnp.zeros((L,), jnp.float32)
    for k in range(1, 128 // L):
        acc_v[pl.ds(k * L, L)] = zeros
    pltpu.sync_copy(acc_v, loss_hbm.at[pl.ds(wid * 128, 128)])


@functools.cache
def _sc_gather():
    return pl.kernel(
        _sc_body,
        out_type=(
            jax.ShapeDtypeStruct((N_TOK * VOCAB,), jnp.float32),
            jax.ShapeDtypeStruct((NW * 128,), jnp.float32),
        ),
        mesh=plsc.VectorSubcoreMesh(core_axis_name="c", subcore_axis_name="s"),
        compiler_params=pltpu.CompilerParams(needs_layout_passes=False),
        scratch_types=(
            pltpu.VMEM((TPW,), jnp.int32),
            pltpu.VMEM((TPW,), jnp.int32),
            pltpu.VMEM((VPAD,), jnp.float32),
            pltpu.VMEM((C, VPAD // 128, 128), jnp.float32),
            pltpu.VMEM((C, VPAD // 128, 128), jnp.float32),
            pltpu.VMEM((CW,), jnp.float32),
            pltpu.VMEM((CW,), jnp.float32),
            pltpu.VMEM((128,), jnp.float32),
            pltpu.SemaphoreType.DMA,
            pltpu.SemaphoreType.DMA,
            pltpu.SemaphoreType.DMA,
            pltpu.SemaphoreType.DMA,
        ),
    )


def kernel(idx, targets, table):
    b, t = idx.shape
    idx_f = idx.reshape(-1)
    tgt_f = targets.reshape(-1)
    lse, padded = _lse_and_pad(table)
    out_flat, loss_part = _sc_gather()(idx_f, tgt_f, lse, padded)
    logits = out_flat.reshape(b, t, VOCAB)
    loss = jnp.sum(loss_part) / float(N_TOK)
    return (logits, loss)


# speedup vs baseline: 2.2341x; 2.2341x over previous
import functools

import jax
import jax.numpy as jnp
from jax import lax
from jax.experimental import pallas as pl
from jax.experimental.pallas import tpu as pltpu
from jax.experimental.pallas import tpu_sc as plsc

VOCAB = 1000
VPAD = 1024
NC, NS, L = 2, 16, 16
NW = NC * NS
N_TOK = 16 * 2048
TPW = N_TOK // NW
C = 16
NCH = TPW // C
CW = C * VOCAB


def _lse_pad_body(table_ref, lse_ref, padded_ref):
    x = table_ref[...]
    m = jnp.max(x, axis=1, keepdims=True)
    s = jnp.sum(jnp.exp(x - m), axis=1, keepdims=True)
    lse_ref[...] = m + jnp.log(s)
    padded_ref[...] = jnp.concatenate(
        [x, jnp.zeros((VOCAB, VPAD - VOCAB), jnp.float32)], axis=1)


def _lse_and_pad(table):
    lse2d, padded = pl.pallas_call(
        _lse_pad_body,
        out_shape=(
            jax.ShapeDtypeStruct((VOCAB, 1), jnp.float32),
            jax.ShapeDtypeStruct((VOCAB, VPAD), jnp.float32),
        ),
    )(table)
    return (jnp.pad(lse2d.reshape(VOCAB), (0, VPAD - VOCAB)),
            padded.reshape(VOCAB, VPAD // 128, 128))


def _sc_body(idx_hbm, tgt_hbm, lse_hbm, table_hbm, out_hbm, loss_hbm,
             idx_v, tgt_v, lse_v, g_a, g_b, p_a, p_b, acc_v,
             sem_ga, sem_gb, sem_sa, sem_sb):
    wid = lax.axis_index("c") * NS + lax.axis_index("s")
    base = wid * TPW
    pltpu.sync_copy(idx_hbm.at[pl.ds(base, TPW)], idx_v)
    pltpu.sync_copy(tgt_hbm.at[pl.ds(base, TPW)], tgt_v)
    pltpu.sync_copy(lse_hbm, lse_v)
    lane = lax.iota(jnp.int32, L)
    tail_mask = lane < (VOCAB % L)

    def chunk_ids(c):
        return idx_v[pl.ds(c * C, C)]

    def gather_start(c, g, sem):
        pltpu.async_copy(table_hbm.at[chunk_ids(c)], g, sem)

    def gather_wait(c, g, sem):
        pltpu.make_async_copy(table_hbm.at[chunk_ids(c)], g, sem).wait()

    def store_start(c, p, sem):
        pltpu.async_copy(p, out_hbm.at[pl.ds(base + c * C, C)], sem)

    def store_drain(c, p, sem):
        pltpu.make_async_copy(
            p, out_hbm.at[pl.ds(base + c * C, C)], sem).wait()

    def repack(g, p):
        @plsc.parallel_loop(0, C, unroll=2)
        def _(j):
            jvec = jnp.full((L,), 0, jnp.int32) + j
            for k in range(VOCAB // L):
                v = g[j, k // 8, pl.ds((k % 8) * L, L)]
                plsc.store_scatter(p, [jvec, lane + L * k], v)
            v = g[j, 7, pl.ds(96, L)]
            plsc.store_scatter(p, [jvec, lane + (VOCAB - (VOCAB % L))], v,
                               mask=tail_mask)

    def loss_chunk(c, g, acc):
        ids = chunk_ids(c)
        tgs = tgt_v[pl.ds(c * C, C)]
        ls = plsc.load_gather(lse_v, [ids])
        tl = plsc.load_gather(g, [lane, tgs >> 7, tgs & 127])
        return acc + (ls - tl)

    gather_start(0, g_a, sem_ga)

    def body(i2, acc):
        ca = 2 * i2
        cb = ca + 1

        gather_start(cb, g_b, sem_gb)
        gather_wait(ca, g_a, sem_ga)
        acc = loss_chunk(ca, g_a, acc)

        @pl.when(i2 > 0)
        def _():
            store_drain(ca - 2, p_a, sem_sa)

        repack(g_a, p_a)
        store_start(ca, p_a, sem_sa)

        @pl.when(ca + 2 < NCH)
        def _():
            gather_start(ca + 2, g_a, sem_ga)

        gather_wait(cb, g_b, sem_gb)
        acc = loss_chunk(cb, g_b, acc)

        @pl.when(i2 > 0)
        def _():
            store_drain(cb - 2, p_b, sem_sb)

        repack(g_b, p_b)
        store_start(cb, p_b, sem_sb)
        return acc

    acc = lax.fori_loop(0, NCH // 2, body, jnp.zeros((L,), jnp.float32))
    store_drain(NCH - 2, p_a, sem_sa)
    store_drain(NCH - 1, p_b, sem_sb)
    acc_v[pl.ds(0, L)] = acc
    zeros = jnp.zeros((L,), jnp.float32)
    for k in range(1, 128 // L):
        acc_v[pl.ds(k * L, L)] = zeros
    pltpu.sync_copy(acc_v, loss_hbm.at[pl.ds(wid * 128, 128)])


@functools.cache
def _sc_gather():
    return pl.kernel(
        _sc_body,
        out_type=(
            jax.ShapeDtypeStruct((N_TOK, VOCAB), jnp.float32),
            jax.ShapeDtypeStruct((NW * 128,), jnp.float32),
        ),
        mesh=plsc.VectorSubcoreMesh(core_axis_name="c", subcore_axis_name="s"),
        compiler_params=pltpu.CompilerParams(needs_layout_passes=False),
        scratch_types=(
            pltpu.VMEM((TPW,), jnp.int32),
            pltpu.VMEM((TPW,), jnp.int32),
            pltpu.VMEM((VPAD,), jnp.float32),
            pltpu.VMEM((C, VPAD // 128, 128), jnp.float32),
            pltpu.VMEM((C, VPAD // 128, 128), jnp.float32),
            pltpu.VMEM((C, VOCAB), jnp.float32),
            pltpu.VMEM((C, VOCAB), jnp.float32),
            pltpu.VMEM((128,), jnp.float32),
            pltpu.SemaphoreType.DMA,
            pltpu.SemaphoreType.DMA,
            pltpu.SemaphoreType.DMA,
            pltpu.SemaphoreType.DMA,
        ),
    )


def kernel(idx, targets, table):
    b, t = idx.shape
    idx_f = idx.reshape(-1)
    tgt_f = targets.reshape(-1)
    lse, padded = _lse_and_pad(table)
    out_flat, loss_part = _sc_gather()(idx_f, tgt_f, lse, padded)
    logits = out_flat.reshape(b, t, VOCAB)
    loss = jnp.sum(loss_part) / float(N_TOK)
    return (logits, loss)
